# baseline (speedup 1.0000x reference)
import jax
import jax.numpy as jnp
from jax.experimental import pallas as pl
from jax.experimental.pallas import tpu as pltpu

_B = 32
_N = 127
_T = 10
_FEAT = 128
_AGG = 32

_M = [2 ** (6 - t) for t in range(7)]
_LO = [m - 1 for m in _M]
_NW = [_B * m for m in _M]
_OFF = [0]
for _n in _NW:
    _OFF.append(_OFF[-1] + _n)
_R = _OFF[-1]

_CJ = [(13 * j) // 7 for j in range(10)]
_PJ = [(13 * j) % 7 for j in range(10)]
_EN_ROWS = 4144

_POST = []
_stack = [(0, False)]
while _stack:
    _nd, _done = _stack.pop()
    if _done:
        _POST.append(_nd)
    else:
        _stack.append((_nd, True))
        if 2 * _nd + 2 < _N:
            _stack.append((2 * _nd + 2, False))
        if 2 * _nd + 1 < _N:
            _stack.append((2 * _nd + 1, False))


def _tree_body(pool_ref, wenc_ref, benc_ref, wc_ref, bc_ref, ws_ref, bs_ref,
               res_ref):
    enc = jnp.maximum(
        jnp.dot(pool_ref[...], wenc_ref[...],
                preferred_element_type=jnp.float32) + benc_ref[...], 0.0)
    res_ref[...] = (jnp.dot(enc, wc_ref[...],
                            preferred_element_type=jnp.float32) + bc_ref[...])
    ws = ws_ref[...]
    bs2 = 2.0 * bs_ref[...]
    for t in range(1, 7):
        o, n, po = _OFF[t], _NW[t], _OFF[t - 1]
        csum = res_ref[pl.ds(po, n, 2), :] + res_ref[pl.ds(po + 1, n, 2), :]
        res_ref[o:o + n, :] = (
            res_ref[o:o + n, :]
            + jnp.dot(csum, ws, preferred_element_type=jnp.float32) + bs2)


def kernel(emb_table, w_enc_pad, b_enc_pad, w_c_pad, b_c_pad, w_sum_pad,
           b_sum_pad):
    en = emb_table[1:1 + 7 * _EN_ROWS].reshape(_EN_ROWS, 7, _FEAT)
    slabs = []
    for t in range(7):
        lo, m = _LO[t], _M[t]
        acc = None
        for j in range(10):
            s = en[_CJ[j] + lo:_CJ[j] + lo + _B * _N, _PJ[j], :]
            s = s.reshape(_B, _N, _FEAT)[:, :m, :]
            acc = s if acc is None else acc + s
        slabs.append(acc.reshape(_B * m, _FEAT))
    poolg = 0.1 * jnp.concatenate(slabs, axis=0)

    res = pl.pallas_call(
        _tree_body,
        out_shape=jax.ShapeDtypeStruct((_R, _FEAT), jnp.float32),
        compiler_params=pltpu.CompilerParams(vmem_limit_bytes=32 << 20),
    )(poolg, w_enc_pad, b_enc_pad, w_c_pad, b_c_pad, w_sum_pad, b_sum_pad)

    waves = [res[_OFF[t]:_OFF[t] + _NW[t]].reshape(_B, _M[t], _FEAT)
             for t in range(7)]
    cols = []
    for nd in _POST:
        lvl = (nd + 1).bit_length() - 1
        cols.append(waves[6 - lvl][:, nd - _LO[6 - lvl], :])
    stack = jnp.maximum(jnp.stack(cols, axis=0), 0.0)
    node_stack = stack[:, :, :_AGG]
    pooled_out = jnp.max(node_stack, axis=0)
    return node_stack, pooled_out


# speedup vs baseline: 10.6595x; 10.6595x over previous
import jax
import jax.numpy as jnp
from jax.experimental import pallas as pl
from jax.experimental.pallas import tpu as pltpu

_B = 32
_N = 127
_T = 10
_FEAT = 128
_AGG = 32

_M = [2 ** (6 - t) for t in range(7)]
_LO = [m - 1 for m in _M]
_NW = [_B * m for m in _M]
_OFF = [0]
for _n in _NW:
    _OFF.append(_OFF[-1] + _n)
_R = _OFF[-1]

_CJ = [(13 * j) // 7 for j in range(10)]
_PJ = [(13 * j) % 7 for j in range(10)]
_EN_ROWS = 4144

_POST = []
_stack = [(0, False)]
while _stack:
    _nd, _done = _stack.pop()
    if _done:
        _POST.append(_nd)
    else:
        _stack.append((_nd, True))
        if 2 * _nd + 2 < _N:
            _stack.append((2 * _nd + 2, False))
        if 2 * _nd + 1 < _N:
            _stack.append((2 * _nd + 1, False))


def _tree_body(pool_ref, wenc_ref, benc_ref, wc_ref, bc_ref, ws_ref, bs_ref,
               res_ref):
    enc = jnp.maximum(
        jnp.dot(pool_ref[...], wenc_ref[...],
                preferred_element_type=jnp.float32) + benc_ref[...], 0.0)
    res_ref[...] = (jnp.dot(enc, wc_ref[...],
                            preferred_element_type=jnp.float32) + bc_ref[...])
    ws = ws_ref[...]
    bs2 = 2.0 * bs_ref[...]
    for t in range(1, 7):
        o, n, po = _OFF[t], _NW[t], _OFF[t - 1]
        csum = res_ref[pl.ds(po, n, 2), :] + res_ref[pl.ds(po + 1, n, 2), :]
        res_ref[o:o + n, :] = (
            res_ref[o:o + n, :]
            + jnp.dot(csum, ws, preferred_element_type=jnp.float32) + bs2)


def kernel(emb_table, w_enc_pad, b_enc_pad, w_c_pad, b_c_pad, w_sum_pad,
           b_sum_pad):
    en = emb_table[1:1 + 7 * _EN_ROWS].reshape(_EN_ROWS, 7, _FEAT)
    acc = None
    for j in range(10):
        s = en[_CJ[j]:_CJ[j] + _B * _N, _PJ[j], :]
        acc = s if acc is None else acc + s
    nat3 = (0.1 * acc).reshape(_B, _N, _FEAT)
    poolg = jnp.concatenate(
        [nat3[:, _LO[t]:_LO[t] + _M[t], :].reshape(_B * _M[t], _FEAT)
         for t in range(7)], axis=0)

    res = pl.pallas_call(
        _tree_body,
        out_shape=jax.ShapeDtypeStruct((_R, _FEAT), jnp.float32),
        compiler_params=pltpu.CompilerParams(vmem_limit_bytes=32 << 20),
    )(poolg, w_enc_pad, b_enc_pad, w_c_pad, b_c_pad, w_sum_pad, b_sum_pad)

    waves = [res[_OFF[t]:_OFF[t] + _NW[t]].reshape(_B, _M[t], _FEAT)
             for t in range(7)]
    cols = []
    for nd in _POST:
        lvl = (nd + 1).bit_length() - 1
        cols.append(waves[6 - lvl][:, nd - _LO[6 - lvl], :])
    stack = jnp.maximum(jnp.stack(cols, axis=0), 0.0)
    node_stack = stack[:, :, :_AGG]
    pooled_out = jnp.max(node_stack, axis=0)
    return node_stack, pooled_out


# speedup vs baseline: 68.6397x; 6.4393x over previous
import jax
import jax.numpy as jnp
from jax.experimental import pallas as pl
from jax.experimental.pallas import tpu as pltpu

_B = 32
_N = 127
_T = 10
_FEAT = 128
_AGG = 32
_RT = _B * _N
_EROWS = 28560

_M = [2 ** (6 - t) for t in range(7)]
_LO = [m - 1 for m in _M]
_NW = [_B * m for m in _M]
_OFF = [0]
for _n in _NW:
    _OFF.append(_OFF[-1] + _n)

_POST = []
_stack = [(0, False)]
while _stack:
    _nd, _done = _stack.pop()
    if _done:
        _POST.append(_nd)
    else:
        _stack.append((_nd, True))
        if 2 * _nd + 2 < _N:
            _stack.append((2 * _nd + 2, False))
        if 2 * _nd + 1 < _N:
            _stack.append((2 * _nd + 1, False))


def _node_tk(nd):
    lvl = (nd + 1).bit_length() - 1
    t = 6 - lvl
    return t, nd - _LO[t]


def _tree_body(emb_ref, wenc_ref, benc_ref, wc_ref, bc_ref, ws_ref, bs_ref,
               out_ref, pmax_ref, nat_ref, res_ref):
    pooled = emb_ref[pl.ds(1, _RT, 7), :]
    for j in range(1, _T):
        pooled = pooled + emb_ref[pl.ds(13 * j + 1, _RT, 7), :]
    pooled = pooled * (1.0 / _T)

    enc = jnp.maximum(
        jnp.dot(pooled, wenc_ref[...],
                preferred_element_type=jnp.float32) + benc_ref[...], 0.0)
    nat_ref[...] = (jnp.dot(enc, wc_ref[...],
                            preferred_element_type=jnp.float32) + bc_ref[...])

    for t in range(7):
        lo, m, o = _LO[t], _M[t], _OFF[t]
        for k in range(m):
            res_ref[o + _B * k:o + _B * (k + 1), :] = (
                nat_ref[pl.ds(lo + k, _B, _N), :])

    ws = ws_ref[...]
    bs2 = 2.0 * bs_ref[...]
    for t in range(1, 7):
        o, n, po = _OFF[t], _NW[t], _OFF[t - 1]
        v = res_ref[po:po + 2 * n, :].reshape(n // _B, 2, _B, _FEAT)
        csum = (v[:, 0] + v[:, 1]).reshape(n, _FEAT)
        res_ref[o:o + n, :] = (
            res_ref[o:o + n, :]
            + jnp.dot(csum, ws, preferred_element_type=jnp.float32) + bs2)

    acc = None
    for idx, nd in enumerate(_POST):
        t, k = _node_tk(nd)
        slab = jnp.maximum(res_ref[_OFF[t] + _B * k:_OFF[t] + _B * (k + 1), :],
                           0.0)
        out_ref[_B * idx:_B * (idx + 1), :] = slab
        acc = slab if acc is None else jnp.maximum(acc, slab)
    pmax_ref[...] = acc


def kernel(emb_table, w_enc_pad, b_enc_pad, w_c_pad, b_c_pad, w_sum_pad,
           b_sum_pad):
    full = lambda s: pl.BlockSpec(s, lambda i: tuple(0 for _ in s))
    out, pmax = pl.pallas_call(
        _tree_body,
        grid=(1,),
        out_shape=(jax.ShapeDtypeStruct((_RT, _FEAT), jnp.float32),
                   jax.ShapeDtypeStruct((_B, _FEAT), jnp.float32)),
        in_specs=[
            pl.BlockSpec((_EROWS, _FEAT), lambda i: (0, 0)),
            full((_FEAT, _FEAT)), full((1, _FEAT)),
            full((_FEAT, _FEAT)), full((1, _FEAT)),
            full((_FEAT, _FEAT)), full((1, _FEAT)),
        ],
        out_specs=(full((_RT, _FEAT)), full((_B, _FEAT))),
        scratch_shapes=[pltpu.VMEM((_RT, _FEAT), jnp.float32),
                        pltpu.VMEM((_RT, _FEAT), jnp.float32)],
        compiler_params=pltpu.CompilerParams(vmem_limit_bytes=48 << 20),
    )(emb_table, w_enc_pad, b_enc_pad, w_c_pad, b_c_pad, w_sum_pad, b_sum_pad)

    node_stack = out.reshape(_N, _B, _FEAT)[:, :, :_AGG]
    pooled_out = pmax[:, :_AGG]
    return node_stack, pooled_out


# speedup vs baseline: 83.9426x; 1.2229x over previous
import jax
import jax.numpy as jnp
from jax.experimental import pallas as pl
from jax.experimental.pallas import tpu as pltpu

_B = 32
_N = 127
_T = 10
_FEAT = 128
_AGG = 32
_RT = _B * _N
_EROWS = 28560

_M = [2 ** (6 - t) for t in range(7)]
_LO = [m - 1 for m in _M]
_NW = [_B * m for m in _M]
_OFF = [0]
for _n in _NW:
    _OFF.append(_OFF[-1] + _n)

_POST = []
_stack = [(0, False)]
while _stack:
    _nd, _done = _stack.pop()
    if _done:
        _POST.append(_nd)
    else:
        _stack.append((_nd, True))
        if 2 * _nd + 2 < _N:
            _stack.append((2 * _nd + 2, False))
        if 2 * _nd + 1 < _N:
            _stack.append((2 * _nd + 1, False))


def _node_tk(nd):
    lvl = (nd + 1).bit_length() - 1
    t = 6 - lvl
    return t, nd - _LO[t]


def _tree_body(emb_ref, w_ref, b_ref, out_ref, pmax_ref, nat_ref, res_ref):
    pooled = emb_ref[pl.ds(1, _RT, 7), :]
    for j in range(1, _T):
        pooled = pooled + emb_ref[pl.ds(13 * j + 1, _RT, 7), :]
    pooled = pooled * (1.0 / _T)

    enc = jnp.maximum(
        jnp.dot(pooled, w_ref[0:_FEAT, :],
                preferred_element_type=jnp.float32) + b_ref[0:1, :], 0.0)
    nat_ref[...] = (jnp.dot(enc, w_ref[_FEAT:2 * _FEAT, :],
                            preferred_element_type=jnp.float32) + b_ref[1:2, :])

    for t in range(7):
        lo, m, o = _LO[t], _M[t], _OFF[t]
        for k in range(m):
            res_ref[o + _B * k:o + _B * (k + 1), :] = (
                nat_ref[pl.ds(lo + k, _B, _N), :])

    ws = w_ref[2 * _FEAT:3 * _FEAT, :]
    bs2 = 2.0 * b_ref[2:3, :]
    for t in range(1, 7):
        o, n, po = _OFF[t], _NW[t], _OFF[t - 1]
        v = res_ref[po:po + 2 * n, :].reshape(n // _B, 2, _B, _FEAT)
        csum = (v[:, 0] + v[:, 1]).reshape(n, _FEAT)
        res_ref[o:o + n, :] = (
            res_ref[o:o + n, :]
            + jnp.dot(csum, ws, preferred_element_type=jnp.float32) + bs2)

    acc = None
    for idx, nd in enumerate(_POST):
        t, k = _node_tk(nd)
        slab = jnp.maximum(res_ref[_OFF[t] + _B * k:_OFF[t] + _B * (k + 1), :],
                           0.0)
        out_ref[idx, :, :] = slab[:, :_AGG]
        acc = slab if acc is None else jnp.maximum(acc, slab)
    pmax_ref[...] = acc[:, :_AGG]


def kernel(emb_table, w_enc_pad, b_enc_pad, w_c_pad, b_c_pad, w_sum_pad,
           b_sum_pad):
    wcat = jnp.concatenate([w_enc_pad, w_c_pad, w_sum_pad], axis=0)
    bcat = jnp.concatenate([b_enc_pad, b_c_pad, b_sum_pad], axis=0)
    node_stack, pooled_out = pl.pallas_call(
        _tree_body,
        grid=(1,),
        out_shape=(jax.ShapeDtypeStruct((_N, _B, _AGG), jnp.float32),
                   jax.ShapeDtypeStruct((_B, _AGG), jnp.float32)),
        in_specs=[
            pl.BlockSpec((_EROWS, _FEAT), lambda i: (0, 0)),
            pl.BlockSpec((3 * _FEAT, _FEAT), lambda i: (0, 0)),
            pl.BlockSpec((3, _FEAT), lambda i: (0, 0)),
        ],
        out_specs=(pl.BlockSpec((_N, _B, _AGG), lambda i: (0, 0, 0)),
                   pl.BlockSpec((_B, _AGG), lambda i: (0, 0))),
        scratch_shapes=[pltpu.VMEM((_RT, _FEAT), jnp.float32),
                        pltpu.VMEM((_RT, _FEAT), jnp.float32)],
        compiler_params=pltpu.CompilerParams(vmem_limit_bytes=48 << 20),
    )(emb_table, wcat, bcat)
    return node_stack, pooled_out
